# baseline (speedup 1.0000x reference)
import functools

import jax
import jax.numpy as jnp
from jax import lax
from jax.experimental import pallas as pl
from jax.experimental.pallas import tpu as pltpu
from jax.experimental.pallas import tpu_sc as plsc

_NA = 256
_NCH = 32
_CAP = _NCH * 16
_ACC = 272
_NW = 32
_RB = 32


def _full_body(h_ref, s_ref, x_ref, o_ref, S_scratch, *, d_in, d_out):
    @pl.when(pl.program_id(0) == 0)
    def _build():
        col = lax.broadcasted_iota(jnp.int32, (d_in, d_out), 1)
        for i in range(3):
            h = h_ref[i, :].reshape(d_in, 1)
            sg = s_ref[i, :].reshape(d_in, 1).astype(jnp.float32)
            S_scratch[i] = jnp.where(col == h, sg, 0.0)

    x = x_ref[...]
    acc = jnp.dot(x, S_scratch[0], preferred_element_type=jnp.float32)
    acc = acc * jnp.dot(x, S_scratch[1], preferred_element_type=jnp.float32)
    acc = acc * jnp.dot(x, S_scratch[2], preferred_element_type=jnp.float32)
    o_ref[...] = acc


def _active_rank(h_ref, d_in, d_out):
    col = lax.broadcasted_iota(jnp.int32, (d_in, d_out), 1)
    pres = []
    for i in range(3):
        m = col == h_ref[i, :].reshape(d_in, 1)
        pres.append(jnp.max(m.astype(jnp.int32), axis=0, keepdims=True))
    active = pres[0] * pres[1] * pres[2]
    r0 = lax.broadcasted_iota(jnp.int32, (d_out, d_out), 0)
    c0 = lax.broadcasted_iota(jnp.int32, (d_out, d_out), 1)
    tri = (r0 <= c0).astype(jnp.float32)
    rank = (
        jnp.dot(active.astype(jnp.float32), tri, preferred_element_type=jnp.float32)
        .astype(jnp.int32)
        - 1
    )
    return col, active, rank


def _compact_body(h_ref, s_ref, x_ref, o_ref, G0_ref, G1_ref, G2_ref, E_ref, *, d_in, d_out):
    G_refs = (G0_ref, G1_ref, G2_ref)

    @pl.when(pl.program_id(0) == 0)
    def _build():
        col, active, rank = _active_rank(h_ref, d_in, d_out)
        colA = lax.broadcasted_iota(jnp.int32, (d_in, _NA), 1)
        for i in range(3):
            m = col == h_ref[i, :].reshape(d_in, 1)
            mi = m.astype(jnp.int32)
            valid = jnp.sum(mi * active, axis=1, keepdims=True)
            slot = jnp.sum(mi * rank, axis=1, keepdims=True)
            slot = jnp.where(valid > 0, slot, _NA)
            sg = s_ref[i, :].reshape(d_in, 1).astype(jnp.float32)
            G_refs[i][...] = jnp.where(colA == slot, sg, 0.0)
        rowA = lax.broadcasted_iota(jnp.int32, (_NA, d_out), 0)
        E_ref[...] = jnp.where((rowA == rank) & (active > 0), 1.0, 0.0)

    x = x_ref[...]
    y = jnp.dot(x, G0_ref[...], preferred_element_type=jnp.float32)
    y = y * jnp.dot(x, G1_ref[...], preferred_element_type=jnp.float32)
    y = y * jnp.dot(x, G2_ref[...], preferred_element_type=jnp.float32)
    o_ref[...] = jnp.dot(y, E_ref[...], preferred_element_type=jnp.float32)


def _expand_body(h_ref, y_ref, o_ref, E_ref, *, d_in, d_out):
    @pl.when(pl.program_id(0) == 0)
    def _build():
        _, active, rank = _active_rank(h_ref, d_in, d_out)
        rowA = lax.broadcasted_iota(jnp.int32, (_NA, d_out), 0)
        E_ref[...] = jnp.where((rowA == rank) & (active > 0), 1.0, 0.0)

    o_ref[...] = jnp.dot(y_ref[...], E_ref[...], preferred_element_type=jnp.float32)


def _make_sc_call(batch, d_in):
    rows_w = batch // _NW
    nblocks = rows_w // _RB
    acc_stride = 3 * _ACC
    mesh = plsc.VectorSubcoreMesh(core_axis_name="c", subcore_axis_name="s")

    @functools.partial(
        pl.kernel,
        mesh=mesh,
        compiler_params=pltpu.CompilerParams(needs_layout_passes=False),
        out_type=jax.ShapeDtypeStruct((batch * _NA,), jnp.float32),
        scratch_types=[
            pltpu.VMEM((3 * _NCH * 16,), jnp.int32),
            pltpu.VMEM((3 * _NCH * 16,), jnp.int32),
            pltpu.VMEM((3 * _NCH * 16,), jnp.float32),
            pltpu.VMEM((_RB * d_in,), jnp.float32),
            pltpu.VMEM((_RB * acc_stride,), jnp.float32),
            pltpu.VMEM((_RB * _NA,), jnp.float32),
        ],
    )
    def sc_sketch(x_hbm, jidx_hbm, slot_hbm, sgn_hbm, y_hbm,
                  jidx_v, slot_v, sgn_v, xbuf, acc, ybuf):
        wid = lax.axis_index("s") * 2 + lax.axis_index("c")
        base = wid * rows_w
        pltpu.sync_copy(jidx_hbm, jidx_v)
        pltpu.sync_copy(slot_hbm, slot_v)
        pltpu.sync_copy(sgn_hbm, sgn_v)
        zero16 = jnp.zeros((16,), jnp.float32)

        def block_body(b, carry):
            row0 = base + b * _RB
            pltpu.sync_copy(x_hbm.at[pl.ds(row0 * d_in, _RB * d_in)], xbuf)

            def _zero(t, c):
                acc[pl.ds(t * 16, 16)] = zero16
                return c

            lax.fori_loop(0, (_RB * acc_stride) // 16, _zero, 0, unroll=8)

            def ch_body(ch, c):
                iv = jidx_v[pl.ds(ch * 16, 16)]
                sv = slot_v[pl.ds(ch * 16, 16)]
                gv = sgn_v[pl.ds(ch * 16, 16)]

                def _rows(r, c2):
                    xr = plsc.load_gather(xbuf, [iv + r * d_in])
                    plsc.addupdate_scatter(acc, [sv + r * acc_stride], xr * gv)
                    return c2

                lax.fori_loop(0, _RB, _rows, 0, unroll=8)
                return c

            lax.fori_loop(0, 3 * _NCH, ch_body, 0)

            def _prod(t, c):
                r = t // (_NA // 16)
                q = t % (_NA // 16)
                o = r * acc_stride + q * 16
                a0 = acc[pl.ds(o, 16)]
                a1 = acc[pl.ds(o + _ACC, 16)]
                a2 = acc[pl.ds(o + 2 * _ACC, 16)]
                ybuf[pl.ds(t * 16, 16)] = a0 * a1 * a2
                return c

            lax.fori_loop(0, _RB * (_NA // 16), _prod, 0, unroll=8)

            pltpu.sync_copy(ybuf, y_hbm.at[pl.ds(row0 * _NA, _RB * _NA)])
            return carry

        lax.fori_loop(0, nblocks, block_body, 0)

    return sc_sketch


def _build_meta(hash_indices, signs, d_in, d_out):
    pres = [
        jnp.zeros((d_out,), jnp.bool_).at[hash_indices[i]].set(True, mode="drop")
        for i in range(3)
    ]
    active = pres[0] & pres[1] & pres[2]
    n_active = jnp.sum(active.astype(jnp.int32))
    rank = jnp.cumsum(active.astype(jnp.int32)) - 1

    ok = n_active <= _NA
    p = jnp.arange(d_in, dtype=jnp.int32)
    pos = (p % _NCH) * 16 + (p // _NCH)
    lane_of = jnp.arange(_CAP, dtype=jnp.int32) % 16
    jidx_all, slot_all, sgn_all = [], [], []
    for i in range(3):
        h = hash_indices[i]
        contrib = active[h]
        slot = jnp.where(contrib, rank[h], 0)
        key = jnp.where(contrib, slot * d_in + p, _NA * d_in + p)
        perm = jnp.argsort(key)
        cnt = jnp.sum(contrib.astype(jnp.int32))
        ok = ok & (cnt <= _CAP)
        bucket_cnt = jnp.zeros((_NA,), jnp.int32).at[slot].add(
            contrib.astype(jnp.int32), mode="drop"
        )
        ok = ok & (jnp.max(bucket_cnt) <= _NCH)

        valid = p < cnt
        scat = jnp.where(valid & (pos < _CAP), pos, _CAP)
        jidx = jnp.zeros((_CAP,), jnp.int32).at[scat].set(perm, mode="drop")
        slot_arr = (i * _ACC + _NA + lane_of).astype(jnp.int32)
        slot_arr = slot_arr.at[scat].set(slot[perm] + i * _ACC, mode="drop")
        sgn_arr = jnp.zeros((_CAP,), jnp.float32).at[scat].set(
            signs[i][perm].astype(jnp.float32), mode="drop"
        )
        jidx_all.append(jidx.reshape(_NCH, 16))
        slot_all.append(slot_arr.reshape(_NCH, 16))
        sgn_all.append(sgn_arr.reshape(_NCH, 16))

    return (
        jnp.concatenate(jidx_all, axis=0),
        jnp.concatenate(slot_all, axis=0),
        jnp.concatenate(sgn_all, axis=0),
        ok,
    )


def _make_call(body, scratch_shapes, d_in, d_out, batch, tile):
    return pl.pallas_call(
        functools.partial(body, d_in=d_in, d_out=d_out),
        grid=(batch // tile,),
        in_specs=[
            pl.BlockSpec((3, d_in), lambda i: (0, 0)),
            pl.BlockSpec((3, d_in), lambda i: (0, 0)),
            pl.BlockSpec((tile, d_in), lambda i: (i, 0)),
        ],
        out_specs=pl.BlockSpec((tile, d_out), lambda i: (i, 0)),
        out_shape=jax.ShapeDtypeStruct((batch, d_out), jnp.float32),
        scratch_shapes=scratch_shapes,
    )


@functools.partial(jax.jit, static_argnames=("tile",))
def _run(x, hash_indices, signs, tile=512):
    batch, d_in = x.shape
    d_out = min(2048, 2 * d_in)

    compact_call = _make_call(
        _compact_body,
        [
            pltpu.VMEM((d_in, _NA), jnp.float32),
            pltpu.VMEM((d_in, _NA), jnp.float32),
            pltpu.VMEM((d_in, _NA), jnp.float32),
            pltpu.VMEM((_NA, d_out), jnp.float32),
        ],
        d_in, d_out, batch, tile,
    )
    full_call = _make_call(
        _full_body,
        [pltpu.VMEM((3, d_in, d_out), jnp.float32)],
        d_in, d_out, batch, tile,
    )

    jidx, slot, sgn, sc_ok = _build_meta(hash_indices, signs, d_in, d_out)
    pres = [
        jnp.zeros((d_out,), jnp.bool_).at[hash_indices[i]].set(True, mode="drop")
        for i in range(3)
    ]
    n_active = jnp.sum(pres[0] & pres[1] & pres[2])

    def tc_paths():
        return lax.cond(
            n_active <= _NA,
            lambda: compact_call(hash_indices, signs, x),
            lambda: full_call(hash_indices, signs, x),
        )

    if batch % (_NW * _RB) != 0:
        return tc_paths()

    sc_call = _make_sc_call(batch, d_in)
    expand_call = pl.pallas_call(
        functools.partial(_expand_body, d_in=d_in, d_out=d_out),
        grid=(batch // tile,),
        in_specs=[
            pl.BlockSpec((3, d_in), lambda i: (0, 0)),
            pl.BlockSpec((tile, _NA), lambda i: (i, 0)),
        ],
        out_specs=pl.BlockSpec((tile, d_out), lambda i: (i, 0)),
        out_shape=jax.ShapeDtypeStruct((batch, d_out), jnp.float32),
        scratch_shapes=[pltpu.VMEM((_NA, d_out), jnp.float32)],
    )

    def hybrid():
        y = sc_call(x.reshape(-1), jidx.reshape(-1), slot.reshape(-1), sgn.reshape(-1))
        return expand_call(hash_indices, y.reshape(batch, _NA))

    return lax.cond(sc_ok, hybrid, tc_paths)


def kernel(x, hash_indices, signs):
    return _run(x, hash_indices, signs)


# speedup vs baseline: 6.5087x; 6.5087x over previous
import functools

import jax
import jax.numpy as jnp
from jax import lax
from jax.experimental import pallas as pl
from jax.experimental.pallas import tpu as pltpu

_NA = 256


def _full_body(h_ref, s_ref, x_ref, o_ref, S_scratch, *, d_in, d_out):
    @pl.when(pl.program_id(0) == 0)
    def _build():
        col = lax.broadcasted_iota(jnp.int32, (d_in, d_out), 1)
        for i in range(3):
            h = h_ref[i, :].reshape(d_in, 1)
            sg = s_ref[i, :].reshape(d_in, 1).astype(jnp.float32)
            S_scratch[i] = jnp.where(col == h, sg, 0.0)

    x = x_ref[...]
    acc = jnp.dot(x, S_scratch[0], preferred_element_type=jnp.float32)
    acc = acc * jnp.dot(x, S_scratch[1], preferred_element_type=jnp.float32)
    acc = acc * jnp.dot(x, S_scratch[2], preferred_element_type=jnp.float32)
    o_ref[...] = acc


def _compact_body(h_ref, s_ref, x_ref, o_ref, G0_ref, G1_ref, G2_ref, E_ref, *, d_in, d_out):
    G_refs = (G0_ref, G1_ref, G2_ref)

    @pl.when(pl.program_id(0) == 0)
    def _build():
        col = lax.broadcasted_iota(jnp.int32, (d_in, d_out), 1)
        pres = []
        for i in range(3):
            m = col == h_ref[i, :].reshape(d_in, 1)
            pres.append(jnp.max(m.astype(jnp.int32), axis=0, keepdims=True))
        active = pres[0] * pres[1] * pres[2]
        r0 = lax.broadcasted_iota(jnp.int32, (d_out, d_out), 0)
        c0 = lax.broadcasted_iota(jnp.int32, (d_out, d_out), 1)
        tri = (r0 <= c0).astype(jnp.float32)
        rank = (
            jnp.dot(active.astype(jnp.float32), tri, preferred_element_type=jnp.float32)
            .astype(jnp.int32)
            - 1
        )

        colA = lax.broadcasted_iota(jnp.int32, (d_in, _NA), 1)
        for i in range(3):
            m = col == h_ref[i, :].reshape(d_in, 1)
            mi = m.astype(jnp.int32)
            valid = jnp.sum(mi * active, axis=1, keepdims=True)
            slot = jnp.sum(mi * rank, axis=1, keepdims=True)
            slot = jnp.where(valid > 0, slot, _NA)
            sg = s_ref[i, :].reshape(d_in, 1).astype(jnp.float32)
            G_refs[i][...] = jnp.where(colA == slot, sg, 0.0)

        rowA = lax.broadcasted_iota(jnp.int32, (_NA, d_out), 0)
        E_ref[...] = jnp.where((rowA == rank) & (active > 0), 1.0, 0.0)

    x = x_ref[...]
    y = jnp.dot(x, G0_ref[...], preferred_element_type=jnp.float32)
    y = y * jnp.dot(x, G1_ref[...], preferred_element_type=jnp.float32)
    y = y * jnp.dot(x, G2_ref[...], preferred_element_type=jnp.float32)
    o_ref[...] = jnp.dot(y, E_ref[...], preferred_element_type=jnp.float32)


def _make_call(body, scratch_shapes, d_in, d_out, batch, tile):
    return pl.pallas_call(
        functools.partial(body, d_in=d_in, d_out=d_out),
        grid=(batch // tile,),
        in_specs=[
            pl.BlockSpec((3, d_in), lambda i: (0, 0)),
            pl.BlockSpec((3, d_in), lambda i: (0, 0)),
            pl.BlockSpec((tile, d_in), lambda i: (i, 0)),
        ],
        out_specs=pl.BlockSpec((tile, d_out), lambda i: (i, 0)),
        out_shape=jax.ShapeDtypeStruct((batch, d_out), jnp.float32),
        scratch_shapes=scratch_shapes,
    )


@functools.partial(jax.jit, static_argnames=("tile",))
def _run(x, hash_indices, signs, tile=1024):
    batch, d_in = x.shape
    d_out = min(2048, 2 * d_in)

    compact_call = _make_call(
        _compact_body,
        [
            pltpu.VMEM((d_in, _NA), jnp.float32),
            pltpu.VMEM((d_in, _NA), jnp.float32),
            pltpu.VMEM((d_in, _NA), jnp.float32),
            pltpu.VMEM((_NA, d_out), jnp.float32),
        ],
        d_in, d_out, batch, tile,
    )
    full_call = _make_call(
        _full_body,
        [pltpu.VMEM((3, d_in, d_out), jnp.float32)],
        d_in, d_out, batch, min(tile, 512),
    )

    pres = [
        jnp.zeros((d_out,), jnp.bool_).at[hash_indices[i]].set(True, mode="drop")
        for i in range(3)
    ]
    n_active = jnp.sum(pres[0] & pres[1] & pres[2])

    return lax.cond(
        n_active <= _NA,
        lambda: compact_call(hash_indices, signs, x),
        lambda: full_call(hash_indices, signs, x),
    )


def kernel(x, hash_indices, signs):
    return _run(x, hash_indices, signs)
